# Initial kernel scaffold; baseline (speedup 1.0000x reference)
import jax
import jax.numpy as jnp
from jax import lax
from jax.experimental import pallas as pl
from jax.experimental.pallas import tpu as pltpu
from jax.experimental.pallas import tpu_sc as plsc

N = 10000
E = 160000
D = 128
C = 128
NCHUNK = E // C
NSUB = 16
ROWS_PER_TILE = N // NSUB
ITERS = (NCHUNK + NSUB - 1) // NSUB
CNTW = 16


def _edge_accumulate(s, x_hbm, src_hbm, dst_hbm, src_v, dst_v, rows_v, ones_v,
                     acc_sh, cnt_sh, sem):
    def body(k, carry):
        g = k * NSUB + s
        @pl.when(g < NCHUNK)
        def _():
            off = g * C
            pltpu.sync_copy(src_hbm.at[pl.ds(off, C)], src_v)
            pltpu.sync_copy(dst_hbm.at[pl.ds(off, C)], dst_v)
            pltpu.async_copy(x_hbm.at[src_v], rows_v, sem).wait()
            pltpu.sync_copy(rows_v, acc_sh.at[dst_v], add=True)
            pltpu.sync_copy(ones_v, cnt_sh.at[dst_v], add=True)
        return carry
    lax.fori_loop(0, ITERS, body, 0)


def _sc_body(x_user, x_item, su, du, si, di, zrow, zcnt, ones,
             s_item, cnt_item, s_user, cnt_user,
             acc_sh, cnt_sh, src_v, dst_v, rows_v, ones_v, sem):
    c = lax.axis_index("c")
    s = lax.axis_index("s")
    base = s * ROWS_PER_TILE

    pltpu.sync_copy(zrow, acc_sh.at[pl.ds(base, ROWS_PER_TILE)])
    pltpu.sync_copy(zcnt, cnt_sh.at[pl.ds(base, ROWS_PER_TILE)])
    pltpu.sync_copy(ones, ones_v)
    plsc.subcore_barrier()

    @pl.when(c == 0)
    def _():
        _edge_accumulate(s, x_user, su, du, src_v, dst_v, rows_v, ones_v,
                         acc_sh, cnt_sh, sem)

    @pl.when(c == 1)
    def _():
        _edge_accumulate(s, x_item, si, di, src_v, dst_v, rows_v, ones_v,
                         acc_sh, cnt_sh, sem)

    plsc.subcore_barrier()

    @pl.when(c == 0)
    def _():
        pltpu.sync_copy(acc_sh.at[pl.ds(base, ROWS_PER_TILE)],
                        s_item.at[pl.ds(base, ROWS_PER_TILE)])
        pltpu.sync_copy(cnt_sh.at[pl.ds(base, ROWS_PER_TILE)],
                        cnt_item.at[pl.ds(base, ROWS_PER_TILE)])

    @pl.when(c == 1)
    def _():
        pltpu.sync_copy(acc_sh.at[pl.ds(base, ROWS_PER_TILE)],
                        s_user.at[pl.ds(base, ROWS_PER_TILE)])
        pltpu.sync_copy(cnt_sh.at[pl.ds(base, ROWS_PER_TILE)],
                        cnt_user.at[pl.ds(base, ROWS_PER_TILE)])


def _sc_segment_sums(x_user, x_item, su, du, si, di):
    zrow = jnp.zeros((ROWS_PER_TILE, D), jnp.float32)
    zcnt = jnp.zeros((ROWS_PER_TILE, CNTW), jnp.float32)
    ones = jnp.ones((C, CNTW), jnp.float32)
    f = pl.kernel(
        _sc_body,
        out_type=(
            jax.ShapeDtypeStruct((N, D), jnp.float32),
            jax.ShapeDtypeStruct((N, CNTW), jnp.float32),
            jax.ShapeDtypeStruct((N, D), jnp.float32),
            jax.ShapeDtypeStruct((N, CNTW), jnp.float32),
        ),
        mesh=plsc.VectorSubcoreMesh(core_axis_name="c", subcore_axis_name="s"),
        scratch_types=[
            pltpu.VMEM_SHARED((N, D), jnp.float32),
            pltpu.VMEM_SHARED((N, CNTW), jnp.float32),
            pltpu.VMEM((C,), jnp.int32),
            pltpu.VMEM((C,), jnp.int32),
            pltpu.VMEM((C, D), jnp.float32),
            pltpu.VMEM((C, CNTW), jnp.float32),
            pltpu.SemaphoreType.DMA,
        ],
    )
    return f(x_user, x_item, su, du, si, di, zrow, zcnt, ones)


def _dense_body(s_ref, cnt_ref, x_ref, wl_ref, bl_ref, wr_ref, g_ref, b_ref,
                o_ref):
    cnt = cnt_ref[:, 0:1]
    mean = s_ref[...] / jnp.maximum(cnt, 1.0)
    h = lax.dot_general(mean, wl_ref[...], (((1,), (1,)), ((), ())),
                        preferred_element_type=jnp.float32)
    h = h + lax.dot_general(x_ref[...], wr_ref[...], (((1,), (1,)), ((), ())),
                            preferred_element_type=jnp.float32)
    h = h + bl_ref[...]
    nrm = jnp.sqrt(jnp.sum(h * h, axis=-1, keepdims=True))
    h = h / jnp.maximum(nrm, 1e-12)
    y = h + x_ref[...]
    m = jnp.mean(y, axis=-1, keepdims=True)
    v = jnp.mean((y - m) ** 2, axis=-1, keepdims=True)
    o_ref[...] = jnp.maximum(
        (y - m) * lax.rsqrt(v + 1e-5) * g_ref[...] + b_ref[...], 0.0)


def _dense_tail(seg, cnt, x, Wl, bl, Wr, g, b):
    R = 1000
    return pl.pallas_call(
        _dense_body,
        grid=(N // R,),
        in_specs=[
            pl.BlockSpec((R, D), lambda i: (i, 0)),
            pl.BlockSpec((R, CNTW), lambda i: (i, 0)),
            pl.BlockSpec((R, D), lambda i: (i, 0)),
            pl.BlockSpec((D, D), lambda i: (0, 0)),
            pl.BlockSpec((1, D), lambda i: (0, 0)),
            pl.BlockSpec((D, D), lambda i: (0, 0)),
            pl.BlockSpec((1, D), lambda i: (0, 0)),
            pl.BlockSpec((1, D), lambda i: (0, 0)),
        ],
        out_specs=pl.BlockSpec((R, D), lambda i: (i, 0)),
        out_shape=jax.ShapeDtypeStruct((N, D), jnp.float32),
    )(seg, cnt, x, Wl, bl.reshape(1, D), Wr, g.reshape(1, D), b.reshape(1, D))


def kernel(x_user, x_item, Wl_u2i, bl_u2i, Wr_u2i, Wl_i2u, bl_i2u, Wr_i2u,
           g_user, beta_user, g_item, beta_item, edge_index_u2i,
           edge_index_i2u):
    su = edge_index_u2i[0].astype(jnp.int32)
    du = edge_index_u2i[1].astype(jnp.int32)
    si = edge_index_i2u[0].astype(jnp.int32)
    di = edge_index_i2u[1].astype(jnp.int32)
    s_item, cnt_item, s_user, cnt_user = _sc_segment_sums(
        x_user, x_item, su, du, si, di)
    out_item = _dense_tail(s_item, cnt_item, x_item, Wl_u2i, bl_u2i, Wr_u2i,
                           g_item, beta_item)
    out_user = _dense_tail(s_user, cnt_user, x_user, Wl_i2u, bl_i2u, Wr_i2u,
                           g_user, beta_user)
    return (out_user, out_item)


# speedup vs baseline: 3.4364x; 3.4364x over previous
import jax
import jax.numpy as jnp
from jax import lax
from jax.experimental import pallas as pl
from jax.experimental.pallas import tpu as pltpu
from jax.experimental.pallas import tpu_sc as plsc

N = 10000
E = 160000
D = 128
C = 64
NCHUNK = E // C
NSUB = 16
ROWS_PER_TILE = 632
NPAD = NSUB * ROWS_PER_TILE
ITERS = (NCHUNK + NSUB - 1) // NSUB


def _edge_accumulate(s, x_hbm, src_hbm, dst_hbm, src_v, dst_v, rows_v,
                     acc_sh, sem):
    def body(k, carry):
        g = k * NSUB + s
        @pl.when(g < NCHUNK)
        def _():
            off = g * C
            pltpu.sync_copy(src_hbm.at[pl.ds(off, C)], src_v)
            pltpu.sync_copy(dst_hbm.at[pl.ds(off, C)], dst_v)
            pltpu.async_copy(x_hbm.at[src_v], rows_v, sem).wait()
            pltpu.sync_copy(rows_v, acc_sh.at[dst_v], add=True)
        return carry
    lax.fori_loop(0, ITERS, body, 0)


def _edge_count(s, dst_hbm, dst_v, ones_v, acc_sh):
    def body(k, carry):
        g = k * NSUB + s
        @pl.when(g < NCHUNK)
        def _():
            off = g * C
            pltpu.sync_copy(dst_hbm.at[pl.ds(off, C)], dst_v)
            pltpu.sync_copy(ones_v, acc_sh.at[dst_v], add=True)
        return carry
    lax.fori_loop(0, ITERS, body, 0)


def _zero_acc(base, rows_v, acc_sh):
    for j in range(0, ROWS_PER_TILE, C):
        r = min(C, ROWS_PER_TILE - j)
        pltpu.sync_copy(rows_v.at[pl.ds(0, r)], acc_sh.at[pl.ds(base + j, r)])


def _writeout_acc(base, rows_v, acc_sh, out):
    for j in range(0, ROWS_PER_TILE, C):
        r = min(C, ROWS_PER_TILE - j)
        pltpu.sync_copy(acc_sh.at[pl.ds(base + j, r)], rows_v.at[pl.ds(0, r)])
        pltpu.sync_copy(rows_v.at[pl.ds(0, r)], out.at[pl.ds(base + j, r)])


def _sc_body(x_user, x_item, su, du, si, di, zrow, ones,
             s_item, cnt_item, s_user, cnt_user,
             acc_sh, src_v, dst_v, rows_v, sem):
    c = lax.axis_index("c")
    s = lax.axis_index("s")
    base = s * ROWS_PER_TILE

    pltpu.sync_copy(zrow, rows_v)
    _zero_acc(base, rows_v, acc_sh)
    plsc.subcore_barrier()

    @pl.when(c == 0)
    def _():
        _edge_accumulate(s, x_user, su, du, src_v, dst_v, rows_v, acc_sh, sem)

    @pl.when(c == 1)
    def _():
        _edge_accumulate(s, x_item, si, di, src_v, dst_v, rows_v, acc_sh, sem)

    plsc.subcore_barrier()

    @pl.when(c == 0)
    def _():
        _writeout_acc(base, rows_v, acc_sh, s_item)

    @pl.when(c == 1)
    def _():
        _writeout_acc(base, rows_v, acc_sh, s_user)

    pltpu.sync_copy(zrow, rows_v)
    _zero_acc(base, rows_v, acc_sh)
    pltpu.sync_copy(ones, rows_v)
    plsc.subcore_barrier()

    @pl.when(c == 0)
    def _():
        _edge_count(s, du, dst_v, rows_v, acc_sh)

    @pl.when(c == 1)
    def _():
        _edge_count(s, di, dst_v, rows_v, acc_sh)

    plsc.subcore_barrier()

    @pl.when(c == 0)
    def _():
        _writeout_acc(base, rows_v, acc_sh, cnt_item)

    @pl.when(c == 1)
    def _():
        _writeout_acc(base, rows_v, acc_sh, cnt_user)


def _sc_segment_sums(x_user, x_item, su, du, si, di):
    zrow = jnp.zeros((C, D), jnp.float32)
    ones = jnp.ones((C, D), jnp.float32)
    f = pl.kernel(
        _sc_body,
        out_type=(
            jax.ShapeDtypeStruct((NPAD, D), jnp.float32),
            jax.ShapeDtypeStruct((NPAD, D), jnp.float32),
            jax.ShapeDtypeStruct((NPAD, D), jnp.float32),
            jax.ShapeDtypeStruct((NPAD, D), jnp.float32),
        ),
        mesh=plsc.VectorSubcoreMesh(core_axis_name="c", subcore_axis_name="s"),
        scratch_types=[
            pltpu.VMEM_SHARED((NPAD, D), jnp.float32),
            pltpu.VMEM((C,), jnp.int32),
            pltpu.VMEM((C,), jnp.int32),
            pltpu.VMEM((C, D), jnp.float32),
            pltpu.SemaphoreType.DMA,
        ],
    )
    return f(x_user, x_item, su, du, si, di, zrow, ones)


def _dense_body(s_ref, cnt_ref, x_ref, wl_ref, bl_ref, wr_ref, g_ref, b_ref,
                o_ref):
    cnt = cnt_ref[:, 0:1]
    mean = s_ref[...] / jnp.maximum(cnt, 1.0)
    h = lax.dot_general(mean, wl_ref[...], (((1,), (1,)), ((), ())),
                        preferred_element_type=jnp.float32)
    h = h + lax.dot_general(x_ref[...], wr_ref[...], (((1,), (1,)), ((), ())),
                            preferred_element_type=jnp.float32)
    h = h + bl_ref[...]
    nrm = jnp.sqrt(jnp.sum(h * h, axis=-1, keepdims=True))
    h = h / jnp.maximum(nrm, 1e-12)
    y = h + x_ref[...]
    m = jnp.mean(y, axis=-1, keepdims=True)
    v = jnp.mean((y - m) ** 2, axis=-1, keepdims=True)
    o_ref[...] = jnp.maximum(
        (y - m) * lax.rsqrt(v + 1e-5) * g_ref[...] + b_ref[...], 0.0)


def _dense_tail(seg, cnt, x, Wl, bl, Wr, g, b):
    R = 1000
    return pl.pallas_call(
        _dense_body,
        grid=(N // R,),
        in_specs=[
            pl.BlockSpec((R, D), lambda i: (i, 0)),
            pl.BlockSpec((R, D), lambda i: (i, 0)),
            pl.BlockSpec((R, D), lambda i: (i, 0)),
            pl.BlockSpec((D, D), lambda i: (0, 0)),
            pl.BlockSpec((1, D), lambda i: (0, 0)),
            pl.BlockSpec((D, D), lambda i: (0, 0)),
            pl.BlockSpec((1, D), lambda i: (0, 0)),
            pl.BlockSpec((1, D), lambda i: (0, 0)),
        ],
        out_specs=pl.BlockSpec((R, D), lambda i: (i, 0)),
        out_shape=jax.ShapeDtypeStruct((N, D), jnp.float32),
    )(seg, cnt, x, Wl, bl.reshape(1, D), Wr, g.reshape(1, D), b.reshape(1, D))


def kernel(x_user, x_item, Wl_u2i, bl_u2i, Wr_u2i, Wl_i2u, bl_i2u, Wr_i2u,
           g_user, beta_user, g_item, beta_item, edge_index_u2i,
           edge_index_i2u):
    su = edge_index_u2i[0].astype(jnp.int32)
    du = edge_index_u2i[1].astype(jnp.int32)
    si = edge_index_i2u[0].astype(jnp.int32)
    di = edge_index_i2u[1].astype(jnp.int32)
    s_item, cnt_item, s_user, cnt_user = _sc_segment_sums(
        x_user, x_item, su, du, si, di)
    out_item = _dense_tail(s_item, cnt_item, x_item, Wl_u2i, bl_u2i, Wr_u2i,
                           g_item, beta_item)
    out_user = _dense_tail(s_user, cnt_user, x_user, Wl_i2u, bl_i2u, Wr_i2u,
                           g_user, beta_user)
    return (out_user, out_item)


# speedup vs baseline: 4.6820x; 1.3625x over previous
import jax
import jax.numpy as jnp
from jax import lax
from jax.experimental import pallas as pl
from jax.experimental.pallas import tpu as pltpu
from jax.experimental.pallas import tpu_sc as plsc

N = 10000
E = 160000
D = 128
C = 128
NCHUNK = E // C
NSUB = 16
ROWS_PER_TILE = 632
NPAD = NSUB * ROWS_PER_TILE
ITERS = (NCHUNK + NSUB - 1) // NSUB


def _edge_accumulate(s, x_hbm, src_hbm, dst_hbm, src_v, dst_v, rows_v,
                     acc_sh, sem):
    def body(k, carry):
        g = k * NSUB + s
        @pl.when(g < NCHUNK)
        def _():
            off = g * C
            pltpu.sync_copy(src_hbm.at[pl.ds(off, C)], src_v)
            pltpu.sync_copy(dst_hbm.at[pl.ds(off, C)], dst_v)
            pltpu.async_copy(x_hbm.at[src_v], rows_v, sem).wait()
            pltpu.sync_copy(rows_v, acc_sh.at[dst_v], add=True)
        return carry
    lax.fori_loop(0, ITERS, body, 0)


def _edge_count(s, dst_hbm, dst_v, ones_v, acc_sh):
    def body(k, carry):
        g = k * NSUB + s
        @pl.when(g < NCHUNK)
        def _():
            off = g * C
            pltpu.sync_copy(dst_hbm.at[pl.ds(off, C)], dst_v)
            pltpu.sync_copy(ones_v, acc_sh.at[dst_v], add=True)
        return carry
    lax.fori_loop(0, ITERS, body, 0)


def _zero_acc(base, rows_v, acc_sh):
    for j in range(0, ROWS_PER_TILE, C):
        r = min(C, ROWS_PER_TILE - j)
        pltpu.sync_copy(rows_v.at[pl.ds(0, r)], acc_sh.at[pl.ds(base + j, r)])


def _writeout_acc(base, rows_v, acc_sh, out):
    for j in range(0, ROWS_PER_TILE, C):
        r = min(C, ROWS_PER_TILE - j)
        pltpu.sync_copy(acc_sh.at[pl.ds(base + j, r)], rows_v.at[pl.ds(0, r)])
        pltpu.sync_copy(rows_v.at[pl.ds(0, r)], out.at[pl.ds(base + j, r)])


def _sc_body(x_user, x_item, su, du, si, di, zrow, ones,
             s_item, cnt_item, s_user, cnt_user,
             acc_sh, src_v, dst_v, rows_v, sem):
    c = lax.axis_index("c")
    s = lax.axis_index("s")
    base = s * ROWS_PER_TILE

    pltpu.sync_copy(zrow, rows_v)
    _zero_acc(base, rows_v, acc_sh)
    plsc.subcore_barrier()

    @pl.when(c == 0)
    def _():
        _edge_accumulate(s, x_user, su, du, src_v, dst_v, rows_v, acc_sh, sem)

    @pl.when(c == 1)
    def _():
        _edge_accumulate(s, x_item, si, di, src_v, dst_v, rows_v, acc_sh, sem)

    plsc.subcore_barrier()

    @pl.when(c == 0)
    def _():
        _writeout_acc(base, rows_v, acc_sh, s_item)

    @pl.when(c == 1)
    def _():
        _writeout_acc(base, rows_v, acc_sh, s_user)

    pltpu.sync_copy(zrow, rows_v)
    _zero_acc(base, rows_v, acc_sh)
    pltpu.sync_copy(ones, rows_v)
    plsc.subcore_barrier()

    @pl.when(c == 0)
    def _():
        _edge_count(s, du, dst_v, rows_v, acc_sh)

    @pl.when(c == 1)
    def _():
        _edge_count(s, di, dst_v, rows_v, acc_sh)

    plsc.subcore_barrier()

    @pl.when(c == 0)
    def _():
        _writeout_acc(base, rows_v, acc_sh, cnt_item)

    @pl.when(c == 1)
    def _():
        _writeout_acc(base, rows_v, acc_sh, cnt_user)


def _sc_segment_sums(x_user, x_item, su, du, si, di):
    zrow = jnp.zeros((C, D), jnp.float32)
    ones = jnp.ones((C, D), jnp.float32)
    f = pl.kernel(
        _sc_body,
        out_type=(
            jax.ShapeDtypeStruct((NPAD, D), jnp.float32),
            jax.ShapeDtypeStruct((NPAD, D), jnp.float32),
            jax.ShapeDtypeStruct((NPAD, D), jnp.float32),
            jax.ShapeDtypeStruct((NPAD, D), jnp.float32),
        ),
        mesh=plsc.VectorSubcoreMesh(core_axis_name="c", subcore_axis_name="s"),
        scratch_types=[
            pltpu.VMEM_SHARED((NPAD, D), jnp.float32),
            pltpu.VMEM((C,), jnp.int32),
            pltpu.VMEM((C,), jnp.int32),
            pltpu.VMEM((C, D), jnp.float32),
            pltpu.SemaphoreType.DMA,
        ],
    )
    return f(x_user, x_item, su, du, si, di, zrow, ones)


def _dense_body(s_ref, cnt_ref, x_ref, wl_ref, bl_ref, wr_ref, g_ref, b_ref,
                o_ref):
    cnt = cnt_ref[:, 0:1]
    mean = s_ref[...] / jnp.maximum(cnt, 1.0)
    h = lax.dot_general(mean, wl_ref[...], (((1,), (1,)), ((), ())),
                        preferred_element_type=jnp.float32)
    h = h + lax.dot_general(x_ref[...], wr_ref[...], (((1,), (1,)), ((), ())),
                            preferred_element_type=jnp.float32)
    h = h + bl_ref[...]
    nrm = jnp.sqrt(jnp.sum(h * h, axis=-1, keepdims=True))
    h = h / jnp.maximum(nrm, 1e-12)
    y = h + x_ref[...]
    m = jnp.mean(y, axis=-1, keepdims=True)
    v = jnp.mean((y - m) ** 2, axis=-1, keepdims=True)
    o_ref[...] = jnp.maximum(
        (y - m) * lax.rsqrt(v + 1e-5) * g_ref[...] + b_ref[...], 0.0)


def _dense_tail(seg, cnt, x, Wl, bl, Wr, g, b):
    R = 1000
    return pl.pallas_call(
        _dense_body,
        grid=(N // R,),
        in_specs=[
            pl.BlockSpec((R, D), lambda i: (i, 0)),
            pl.BlockSpec((R, D), lambda i: (i, 0)),
            pl.BlockSpec((R, D), lambda i: (i, 0)),
            pl.BlockSpec((D, D), lambda i: (0, 0)),
            pl.BlockSpec((1, D), lambda i: (0, 0)),
            pl.BlockSpec((D, D), lambda i: (0, 0)),
            pl.BlockSpec((1, D), lambda i: (0, 0)),
            pl.BlockSpec((1, D), lambda i: (0, 0)),
        ],
        out_specs=pl.BlockSpec((R, D), lambda i: (i, 0)),
        out_shape=jax.ShapeDtypeStruct((N, D), jnp.float32),
    )(seg, cnt, x, Wl, bl.reshape(1, D), Wr, g.reshape(1, D), b.reshape(1, D))


def kernel(x_user, x_item, Wl_u2i, bl_u2i, Wr_u2i, Wl_i2u, bl_i2u, Wr_i2u,
           g_user, beta_user, g_item, beta_item, edge_index_u2i,
           edge_index_i2u):
    su = edge_index_u2i[0].astype(jnp.int32)
    du = edge_index_u2i[1].astype(jnp.int32)
    si = edge_index_i2u[0].astype(jnp.int32)
    di = edge_index_i2u[1].astype(jnp.int32)
    s_item, cnt_item, s_user, cnt_user = _sc_segment_sums(
        x_user, x_item, su, du, si, di)
    out_item = _dense_tail(s_item, cnt_item, x_item, Wl_u2i, bl_u2i, Wr_u2i,
                           g_item, beta_item)
    out_user = _dense_tail(s_user, cnt_user, x_user, Wl_i2u, bl_i2u, Wr_i2u,
                           g_user, beta_user)
    return (out_user, out_item)


# speedup vs baseline: 5.8756x; 1.2549x over previous
import jax
import jax.numpy as jnp
from jax import lax
from jax.experimental import pallas as pl
from jax.experimental.pallas import tpu as pltpu
from jax.experimental.pallas import tpu_sc as plsc

N = 10000
E = 160000
D = 128
C = 128
NCHUNK = E // C
NSUB = 16
ROWS_PER_TILE = 632
NPAD = NSUB * ROWS_PER_TILE
ITERS = (NCHUNK + NSUB - 1) // NSUB


def _edge_accumulate(s, x_hbm, src_hbm, dst_hbm, srcb, dstall, rows, acc_sh,
                     semG):
    pltpu.sync_copy(src_hbm.at[pl.ds(s * C, C)], srcb[0])
    pltpu.sync_copy(dst_hbm.at[pl.ds(s * C, C)], dstall.at[0])
    pltpu.async_copy(x_hbm.at[srcb[0]], rows[0], semG)

    def body(kk, carry):
        for b in (0, 1):
            k = 2 * kk + b
            g = k * NSUB + s
            @pl.when(g < NCHUNK)
            def _():
                pltpu.make_async_copy(x_hbm.at[srcb[b]], rows[b], semG).wait()
                @pl.when(g + NSUB < NCHUNK)
                def _():
                    off = (g + NSUB) * C
                    pltpu.sync_copy(src_hbm.at[pl.ds(off, C)], srcb[1 - b])
                    pltpu.sync_copy(dst_hbm.at[pl.ds(off, C)],
                                    dstall.at[k + 1])
                    pltpu.async_copy(x_hbm.at[srcb[1 - b]], rows[1 - b], semG)
                pltpu.sync_copy(rows[b], acc_sh.at[dstall.at[k]], add=True)
        return carry
    lax.fori_loop(0, (ITERS + 1) // 2, body, 0)


def _edge_count(s, dstall, ones_v, acc_sh, semA):
    def body(kk, carry):
        for b in (0, 1):
            k = 2 * kk + b
            g = k * NSUB + s
            @pl.when(g < NCHUNK)
            def _():
                pltpu.async_copy(ones_v, acc_sh.at[dstall.at[k]], semA,
                                 add=True)
                @pl.when(k >= 2)
                def _():
                    pltpu.make_async_copy(ones_v, acc_sh.at[dstall.at[0]],
                                          semA).wait()
        return carry
    lax.fori_loop(0, (ITERS + 1) // 2, body, 0)
    pltpu.make_async_copy(ones_v, acc_sh.at[dstall.at[0]], semA).wait()
    pltpu.make_async_copy(ones_v, acc_sh.at[dstall.at[0]], semA).wait()


def _zero_acc(base, rows_v, acc_sh):
    for j in range(0, ROWS_PER_TILE, C):
        r = min(C, ROWS_PER_TILE - j)
        pltpu.sync_copy(rows_v.at[pl.ds(0, r)], acc_sh.at[pl.ds(base + j, r)])


def _writeout_acc(base, rows_v, acc_sh, out):
    for j in range(0, ROWS_PER_TILE, C):
        r = min(C, ROWS_PER_TILE - j)
        pltpu.sync_copy(acc_sh.at[pl.ds(base + j, r)], rows_v.at[pl.ds(0, r)])
        pltpu.sync_copy(rows_v.at[pl.ds(0, r)], out.at[pl.ds(base + j, r)])


def _sc_body(x_user, x_item, su, du, si, di, zrow, ones,
             s_item, cnt_item, s_user, cnt_user,
             acc_sh, dstall, src0, src1, rows0, rows1, semG, semA):
    c = lax.axis_index("c")
    s = lax.axis_index("s")
    base = s * ROWS_PER_TILE
    srcb = (src0, src1)
    rows = (rows0, rows1)

    pltpu.sync_copy(zrow, rows0)
    _zero_acc(base, rows0, acc_sh)
    plsc.subcore_barrier()

    @pl.when(c == 0)
    def _():
        _edge_accumulate(s, x_user, su, du, srcb, dstall, rows, acc_sh, semG)

    @pl.when(c == 1)
    def _():
        _edge_accumulate(s, x_item, si, di, srcb, dstall, rows, acc_sh, semG)

    plsc.subcore_barrier()

    @pl.when(c == 0)
    def _():
        _writeout_acc(base, rows0, acc_sh, s_item)

    @pl.when(c == 1)
    def _():
        _writeout_acc(base, rows0, acc_sh, s_user)

    pltpu.sync_copy(zrow, rows0)
    _zero_acc(base, rows0, acc_sh)
    pltpu.sync_copy(ones, rows0)
    plsc.subcore_barrier()

    _edge_count(s, dstall, rows0, acc_sh, semA)

    plsc.subcore_barrier()

    @pl.when(c == 0)
    def _():
        _writeout_acc(base, rows1, acc_sh, cnt_item)

    @pl.when(c == 1)
    def _():
        _writeout_acc(base, rows1, acc_sh, cnt_user)


def _sc_segment_sums(x_user, x_item, su, du, si, di):
    zrow = jnp.zeros((C, D), jnp.float32)
    ones = jnp.ones((C, D), jnp.float32)
    f = pl.kernel(
        _sc_body,
        out_type=(
            jax.ShapeDtypeStruct((NPAD, D), jnp.float32),
            jax.ShapeDtypeStruct((NPAD, D), jnp.float32),
            jax.ShapeDtypeStruct((NPAD, D), jnp.float32),
            jax.ShapeDtypeStruct((NPAD, D), jnp.float32),
        ),
        mesh=plsc.VectorSubcoreMesh(core_axis_name="c", subcore_axis_name="s"),
        scratch_types=[
            pltpu.VMEM_SHARED((NPAD, D), jnp.float32),
            pltpu.VMEM((ITERS, C), jnp.int32),
            pltpu.VMEM((C,), jnp.int32),
            pltpu.VMEM((C,), jnp.int32),
            pltpu.VMEM((C, D), jnp.float32),
            pltpu.VMEM((C, D), jnp.float32),
            pltpu.SemaphoreType.DMA,
            pltpu.SemaphoreType.DMA,
        ],
    )
    return f(x_user, x_item, su, du, si, di, zrow, ones)


def _dense_body(s_ref, cnt_ref, x_ref, wl_ref, bl_ref, wr_ref, g_ref, b_ref,
                o_ref):
    cnt = cnt_ref[:, 0:1]
    mean = s_ref[...] / jnp.maximum(cnt, 1.0)
    h = lax.dot_general(mean, wl_ref[...], (((1,), (1,)), ((), ())),
                        preferred_element_type=jnp.float32)
    h = h + lax.dot_general(x_ref[...], wr_ref[...], (((1,), (1,)), ((), ())),
                            preferred_element_type=jnp.float32)
    h = h + bl_ref[...]
    nrm = jnp.sqrt(jnp.sum(h * h, axis=-1, keepdims=True))
    h = h / jnp.maximum(nrm, 1e-12)
    y = h + x_ref[...]
    m = jnp.mean(y, axis=-1, keepdims=True)
    v = jnp.mean((y - m) ** 2, axis=-1, keepdims=True)
    o_ref[...] = jnp.maximum(
        (y - m) * lax.rsqrt(v + 1e-5) * g_ref[...] + b_ref[...], 0.0)


def _dense_tail(seg, cnt, x, Wl, bl, Wr, g, b):
    R = 1000
    return pl.pallas_call(
        _dense_body,
        grid=(N // R,),
        in_specs=[
            pl.BlockSpec((R, D), lambda i: (i, 0)),
            pl.BlockSpec((R, D), lambda i: (i, 0)),
            pl.BlockSpec((R, D), lambda i: (i, 0)),
            pl.BlockSpec((D, D), lambda i: (0, 0)),
            pl.BlockSpec((1, D), lambda i: (0, 0)),
            pl.BlockSpec((D, D), lambda i: (0, 0)),
            pl.BlockSpec((1, D), lambda i: (0, 0)),
            pl.BlockSpec((1, D), lambda i: (0, 0)),
        ],
        out_specs=pl.BlockSpec((R, D), lambda i: (i, 0)),
        out_shape=jax.ShapeDtypeStruct((N, D), jnp.float32),
    )(seg, cnt, x, Wl, bl.reshape(1, D), Wr, g.reshape(1, D), b.reshape(1, D))


def kernel(x_user, x_item, Wl_u2i, bl_u2i, Wr_u2i, Wl_i2u, bl_i2u, Wr_i2u,
           g_user, beta_user, g_item, beta_item, edge_index_u2i,
           edge_index_i2u):
    su = edge_index_u2i[0].astype(jnp.int32)
    du = edge_index_u2i[1].astype(jnp.int32)
    si = edge_index_i2u[0].astype(jnp.int32)
    di = edge_index_i2u[1].astype(jnp.int32)
    s_item, cnt_item, s_user, cnt_user = _sc_segment_sums(
        x_user, x_item, su, du, si, di)
    out_item = _dense_tail(s_item, cnt_item, x_item, Wl_u2i, bl_u2i, Wr_u2i,
                           g_item, beta_item)
    out_user = _dense_tail(s_user, cnt_user, x_user, Wl_i2u, bl_i2u, Wr_i2u,
                           g_user, beta_user)
    return (out_user, out_item)


# speedup vs baseline: 7.4865x; 1.2742x over previous
import jax
import jax.numpy as jnp
from jax import lax
from jax.experimental import pallas as pl
from jax.experimental.pallas import tpu as pltpu
from jax.experimental.pallas import tpu_sc as plsc

N = 10000
E = 160000
D = 128
C = 128
NCHUNK = E // C
NSUB = 16
ROWS_PER_TILE = 632
NPAD = NSUB * ROWS_PER_TILE
ITERS = (NCHUNK + NSUB - 1) // NSUB


def _edge_accumulate(s, x_hbm, src_hbm, dst_hbm, srcb, dstall, rows, acc_sh,
                     semG, semIs):
    pltpu.sync_copy(src_hbm.at[pl.ds(s * C, C)], srcb[0])
    pltpu.sync_copy(dst_hbm.at[pl.ds(s * C, C)], dstall.at[0])
    @pl.when(s + NSUB < NCHUNK)
    def _():
        off = (s + NSUB) * C
        pltpu.async_copy(src_hbm.at[pl.ds(off, C)], srcb[1], semIs[1])
        pltpu.async_copy(dst_hbm.at[pl.ds(off, C)], dstall.at[1], semIs[1])
    pltpu.async_copy(x_hbm.at[srcb[0]], rows[0], semG)

    def body(kk, carry):
        for b in (0, 1):
            k = 2 * kk + b
            g = k * NSUB + s
            @pl.when(g < NCHUNK)
            def _():
                pltpu.make_async_copy(x_hbm.at[srcb[b]], rows[b], semG).wait()
                @pl.when(g + 2 * NSUB < NCHUNK)
                def _():
                    off = (g + 2 * NSUB) * C
                    pltpu.async_copy(src_hbm.at[pl.ds(off, C)], srcb[b],
                                     semIs[b])
                    pltpu.async_copy(dst_hbm.at[pl.ds(off, C)],
                                     dstall.at[k + 2], semIs[b])
                @pl.when(g + NSUB < NCHUNK)
                def _():
                    off = (g + NSUB) * C
                    pltpu.make_async_copy(src_hbm.at[pl.ds(off, C)],
                                          srcb[1 - b], semIs[1 - b]).wait()
                    pltpu.make_async_copy(dst_hbm.at[pl.ds(off, C)],
                                          dstall.at[k + 1], semIs[1 - b]).wait()
                    pltpu.async_copy(x_hbm.at[srcb[1 - b]], rows[1 - b], semG)
                pltpu.sync_copy(rows[b], acc_sh.at[dstall.at[k]], add=True)
        return carry
    lax.fori_loop(0, (ITERS + 1) // 2, body, 0)


def _edge_count(s, dstall, ones_v, acc_sh, semA):
    def body(kk, carry):
        for b in (0, 1):
            k = 2 * kk + b
            g = k * NSUB + s
            @pl.when(g < NCHUNK)
            def _():
                pltpu.async_copy(ones_v, acc_sh.at[dstall.at[k]], semA,
                                 add=True)
                @pl.when(k >= 2)
                def _():
                    pltpu.make_async_copy(ones_v, acc_sh.at[dstall.at[0]],
                                          semA).wait()
        return carry
    lax.fori_loop(0, (ITERS + 1) // 2, body, 0)
    pltpu.make_async_copy(ones_v, acc_sh.at[dstall.at[0]], semA).wait()
    pltpu.make_async_copy(ones_v, acc_sh.at[dstall.at[0]], semA).wait()


def _zero_acc(base, rows_v, acc_sh):
    for j in range(0, ROWS_PER_TILE, C):
        r = min(C, ROWS_PER_TILE - j)
        pltpu.sync_copy(rows_v.at[pl.ds(0, r)], acc_sh.at[pl.ds(base + j, r)])


def _writeout_acc(base, rows_v, acc_sh, out):
    for j in range(0, ROWS_PER_TILE, C):
        r = min(C, ROWS_PER_TILE - j)
        pltpu.sync_copy(acc_sh.at[pl.ds(base + j, r)], rows_v.at[pl.ds(0, r)])
        pltpu.sync_copy(rows_v.at[pl.ds(0, r)], out.at[pl.ds(base + j, r)])


def _sc_body(x_user, x_item, su, du, si, di, zrow, ones,
             s_item, cnt_item, s_user, cnt_user,
             acc_sh, dstall, src0, src1, rows0, rows1, semG, semA, semI0,
             semI1):
    c = lax.axis_index("c")
    s = lax.axis_index("s")
    base = s * ROWS_PER_TILE
    srcb = (src0, src1)
    rows = (rows0, rows1)
    semIs = (semI0, semI1)

    pltpu.sync_copy(zrow, rows0)
    _zero_acc(base, rows0, acc_sh)
    plsc.subcore_barrier()

    @pl.when(c == 0)
    def _():
        _edge_accumulate(s, x_user, su, du, srcb, dstall, rows, acc_sh,
                         semG, semIs)

    @pl.when(c == 1)
    def _():
        _edge_accumulate(s, x_item, si, di, srcb, dstall, rows, acc_sh,
                         semG, semIs)

    plsc.subcore_barrier()

    @pl.when(c == 0)
    def _():
        _writeout_acc(base, rows0, acc_sh, s_item)

    @pl.when(c == 1)
    def _():
        _writeout_acc(base, rows0, acc_sh, s_user)

    pltpu.sync_copy(zrow, rows0)
    _zero_acc(base, rows0, acc_sh)
    pltpu.sync_copy(ones, rows0)
    plsc.subcore_barrier()

    _edge_count(s, dstall, rows0, acc_sh, semA)

    plsc.subcore_barrier()

    @pl.when(c == 0)
    def _():
        _writeout_acc(base, rows1, acc_sh, cnt_item)

    @pl.when(c == 1)
    def _():
        _writeout_acc(base, rows1, acc_sh, cnt_user)


def _sc_segment_sums(x_user, x_item, su, du, si, di):
    zrow = jnp.zeros((C, D), jnp.float32)
    ones = jnp.ones((C, D), jnp.float32)
    f = pl.kernel(
        _sc_body,
        out_type=(
            jax.ShapeDtypeStruct((NPAD, D), jnp.float32),
            jax.ShapeDtypeStruct((NPAD, D), jnp.float32),
            jax.ShapeDtypeStruct((NPAD, D), jnp.float32),
            jax.ShapeDtypeStruct((NPAD, D), jnp.float32),
        ),
        mesh=plsc.VectorSubcoreMesh(core_axis_name="c", subcore_axis_name="s"),
        scratch_types=[
            pltpu.VMEM_SHARED((NPAD, D), jnp.float32),
            pltpu.VMEM((ITERS, C), jnp.int32),
            pltpu.VMEM((C,), jnp.int32),
            pltpu.VMEM((C,), jnp.int32),
            pltpu.VMEM((C, D), jnp.float32),
            pltpu.VMEM((C, D), jnp.float32),
            pltpu.SemaphoreType.DMA,
            pltpu.SemaphoreType.DMA,
            pltpu.SemaphoreType.DMA,
            pltpu.SemaphoreType.DMA,
        ],
    )
    return f(x_user, x_item, su, du, si, di, zrow, ones)


def _dense_body(s_ref, cnt_ref, x_ref, wl_ref, bl_ref, wr_ref, g_ref, b_ref,
                o_ref):
    cnt = cnt_ref[:, 0:1]
    mean = s_ref[...] / jnp.maximum(cnt, 1.0)
    h = lax.dot_general(mean, wl_ref[...], (((1,), (1,)), ((), ())),
                        preferred_element_type=jnp.float32)
    h = h + lax.dot_general(x_ref[...], wr_ref[...], (((1,), (1,)), ((), ())),
                            preferred_element_type=jnp.float32)
    h = h + bl_ref[...]
    nrm = jnp.sqrt(jnp.sum(h * h, axis=-1, keepdims=True))
    h = h / jnp.maximum(nrm, 1e-12)
    y = h + x_ref[...]
    m = jnp.mean(y, axis=-1, keepdims=True)
    v = jnp.mean((y - m) ** 2, axis=-1, keepdims=True)
    o_ref[...] = jnp.maximum(
        (y - m) * lax.rsqrt(v + 1e-5) * g_ref[...] + b_ref[...], 0.0)


def _dense_tail(seg, cnt, x, Wl, bl, Wr, g, b):
    R = 1000
    return pl.pallas_call(
        _dense_body,
        grid=(N // R,),
        in_specs=[
            pl.BlockSpec((R, D), lambda i: (i, 0)),
            pl.BlockSpec((R, D), lambda i: (i, 0)),
            pl.BlockSpec((R, D), lambda i: (i, 0)),
            pl.BlockSpec((D, D), lambda i: (0, 0)),
            pl.BlockSpec((1, D), lambda i: (0, 0)),
            pl.BlockSpec((D, D), lambda i: (0, 0)),
            pl.BlockSpec((1, D), lambda i: (0, 0)),
            pl.BlockSpec((1, D), lambda i: (0, 0)),
        ],
        out_specs=pl.BlockSpec((R, D), lambda i: (i, 0)),
        out_shape=jax.ShapeDtypeStruct((N, D), jnp.float32),
    )(seg, cnt, x, Wl, bl.reshape(1, D), Wr, g.reshape(1, D), b.reshape(1, D))


def kernel(x_user, x_item, Wl_u2i, bl_u2i, Wr_u2i, Wl_i2u, bl_i2u, Wr_i2u,
           g_user, beta_user, g_item, beta_item, edge_index_u2i,
           edge_index_i2u):
    su = edge_index_u2i[0].astype(jnp.int32)
    du = edge_index_u2i[1].astype(jnp.int32)
    si = edge_index_i2u[0].astype(jnp.int32)
    di = edge_index_i2u[1].astype(jnp.int32)
    s_item, cnt_item, s_user, cnt_user = _sc_segment_sums(
        x_user, x_item, su, du, si, di)
    out_item = _dense_tail(s_item, cnt_item, x_item, Wl_u2i, bl_u2i, Wr_u2i,
                           g_item, beta_item)
    out_user = _dense_tail(s_user, cnt_user, x_user, Wl_i2u, bl_i2u, Wr_i2u,
                           g_user, beta_user)
    return (out_user, out_item)
